# baseline (speedup 1.0000x reference)
import functools
import jax
import jax.numpy as jnp
from jax import lax
from jax.experimental import pallas as pl
from jax.experimental.pallas import tpu as pltpu
from jax.experimental.pallas import tpu_sc as plsc

N = 10000
D = 128
HID = 128
OUT = 16
EL = 320000
EG = 80000
H = 4

NC = 2
NS = 16
NW = NC * NS

BN = 1000
NGRID = N // BN
EGP = 81920
EG_PT = EGP // NW

PREC = lax.Precision.HIGHEST


def _dot(a, b):
    return jnp.dot(a, b, precision=PREC)


def _lrelu(v):
    return jnp.where(v > 0, v, 0.2 * v)


def _tc1_body(x_ref, degp_ref, wlp_ref, blp_ref, wgcn_ref, wgat_ref,
              asr_ref, adr_ref,
              xwd_o, xh_o, asrc16_o, adst16_o, eself_o):
    xb = x_ref[...]
    xl = jnp.maximum(_dot(xb, wlp_ref[...]) + blp_ref[...], 0.0)
    deg = jnp.sum(degp_ref[...], axis=1) + 1.0
    dis = lax.rsqrt(deg)
    xw = _dot(xl, wgcn_ref[...])
    xwd_o[...] = dis[:, None] * xw
    xh = _dot(xl, wgat_ref[...])
    xh_o[...] = xh
    acols = []
    bcols = []
    for h in range(H):
        xh_h = xh[:, h * HID:(h + 1) * HID]
        acols.append(_dot(xh_h, asr_ref[...][h][:, None]))
        bcols.append(_dot(xh_h, adr_ref[...][h][:, None]))
    a_src = jnp.concatenate(acols, axis=1)
    a_dst = jnp.concatenate(bcols, axis=1)
    zpad = jnp.zeros((a_src.shape[0], 16 - H), jnp.float32)
    asrc16_o[...] = jnp.concatenate([a_src, zpad], axis=1)
    adst16_o[...] = jnp.concatenate([a_dst, zpad], axis=1)
    eself_o[...] = jnp.exp(_lrelu(a_src + a_dst))


def _tc1(x, deg_part, W_lp, b_lp, W_gcn, W_gat, asrT, adrT):
    return pl.pallas_call(
        _tc1_body,
        grid=(NGRID,),
        in_specs=[
            pl.BlockSpec((BN, D), lambda i: (i, 0)),
            pl.BlockSpec((BN, NW), lambda i: (i, 0)),
            pl.BlockSpec((D, HID), lambda i: (0, 0)),
            pl.BlockSpec((1, HID), lambda i: (0, 0)),
            pl.BlockSpec((HID, HID), lambda i: (0, 0)),
            pl.BlockSpec((HID, H * HID), lambda i: (0, 0)),
            pl.BlockSpec((H, HID), lambda i: (0, 0)),
            pl.BlockSpec((H, HID), lambda i: (0, 0)),
        ],
        out_specs=[
            pl.BlockSpec((BN, HID), lambda i: (i, 0)),
            pl.BlockSpec((BN, H * HID), lambda i: (i, 0)),
            pl.BlockSpec((BN, 16), lambda i: (i, 0)),
            pl.BlockSpec((BN, 16), lambda i: (i, 0)),
            pl.BlockSpec((BN, H), lambda i: (i, 0)),
        ],
        out_shape=[
            jax.ShapeDtypeStruct((N, HID), jnp.float32),
            jax.ShapeDtypeStruct((N, H * HID), jnp.float32),
            jax.ShapeDtypeStruct((N, 16), jnp.float32),
            jax.ShapeDtypeStruct((N, 16), jnp.float32),
            jax.ShapeDtypeStruct((N, H), jnp.float32),
        ],
    )(x, deg_part, W_lp, b_lp, W_gcn, W_gat, asrT, adrT)


def _tc2_body(accp_ref, xwd_ref, degp_ref, bgcn_ref, esump_ref, eself_ref,
              xlg_o, esum16_o, esum4_o):
    deg = jnp.sum(degp_ref[...], axis=1) + 1.0
    dis = lax.rsqrt(deg)
    acc = accp_ref[...][0] + accp_ref[...][1] + xwd_ref[...]
    xlg_o[...] = jnp.maximum(dis[:, None] * acc + bgcn_ref[...], 0.0)
    esum = jnp.sum(esump_ref[...], axis=0) + eself_ref[...]
    esum4_o[...] = esum
    opad = jnp.ones((esum.shape[0], 16 - H), jnp.float32)
    esum16_o[...] = jnp.concatenate([esum, opad], axis=1)


def _tc2(acc_parts, xwd, deg_part, b_gcn, esum_parts, eself):
    return pl.pallas_call(
        _tc2_body,
        grid=(NGRID,),
        in_specs=[
            pl.BlockSpec((NC, BN, HID), lambda i: (0, i, 0)),
            pl.BlockSpec((BN, HID), lambda i: (i, 0)),
            pl.BlockSpec((BN, NW), lambda i: (i, 0)),
            pl.BlockSpec((1, HID), lambda i: (0, 0)),
            pl.BlockSpec((NW, BN, H), lambda i: (0, i, 0)),
            pl.BlockSpec((BN, H), lambda i: (i, 0)),
        ],
        out_specs=[
            pl.BlockSpec((BN, HID), lambda i: (i, 0)),
            pl.BlockSpec((BN, 16), lambda i: (i, 0)),
            pl.BlockSpec((BN, H), lambda i: (i, 0)),
        ],
        out_shape=[
            jax.ShapeDtypeStruct((N, HID), jnp.float32),
            jax.ShapeDtypeStruct((N, 16), jnp.float32),
            jax.ShapeDtypeStruct((N, H), jnp.float32),
        ],
    )(acc_parts, xwd, deg_part, b_gcn, esum_parts, eself)


def _tc3_body(xlg_ref, msgp_ref, xh_ref, eself_ref, esum4_ref, bgat_ref,
              wpa_ref, wpb_ref, bpool_ref, wc1_ref, bc1_ref, wc2_ref,
              bc2_ref, wu_ref, bu_ref,
              pred_o, unc_o, acc_s):
    i = pl.program_id(0)
    att_self = eself_ref[...] / esum4_ref[...]
    xh = xh_ref[...]
    self_msg = att_self[:, 0:1] * xh[:, 0:HID]
    for h in range(1, H):
        self_msg = self_msg + att_self[:, h:h + 1] * xh[:, h * HID:(h + 1) * HID]
    gat = jnp.maximum(
        (msgp_ref[...][0] + msgp_ref[...][1] + self_msg) * (1.0 / H)
        + bgat_ref[...], 0.0)
    pooled = jnp.maximum(
        _dot(xlg_ref[...], wpa_ref[...]) + _dot(gat, wpb_ref[...])
        + bpool_ref[...], 0.0)
    psum = jnp.sum(pooled, axis=0, keepdims=True)

    @pl.when(i == 0)
    def _():
        acc_s[...] = psum

    @pl.when(i > 0)
    def _():
        acc_s[...] = acc_s[...] + psum

    @pl.when(i == NGRID - 1)
    def _():
        xf = acc_s[...] * (1.0 / N)
        hh = jnp.maximum(_dot(xf, wc1_ref[...]) + bc1_ref[...], 0.0)
        pred_o[...] = _dot(hh, wc2_ref[...]) + bc2_ref[...]
        zu = _dot(xf, wu_ref[...]) + bu_ref[...]
        unc_o[...] = 1.0 / (1.0 + jnp.exp(-zu))


def _tc3(xlg, msg_parts, xh, eself, esum4, b_gat, W_pool, b_pool,
         W_c1, b_c1, W_c2, b_c2, W_u, b_u):
    wpa = W_pool[:HID]
    wpb = W_pool[HID:]
    return pl.pallas_call(
        _tc3_body,
        grid=(NGRID,),
        in_specs=[
            pl.BlockSpec((BN, HID), lambda i: (i, 0)),
            pl.BlockSpec((NC, BN, HID), lambda i: (0, i, 0)),
            pl.BlockSpec((BN, H * HID), lambda i: (i, 0)),
            pl.BlockSpec((BN, H), lambda i: (i, 0)),
            pl.BlockSpec((BN, H), lambda i: (i, 0)),
            pl.BlockSpec((1, HID), lambda i: (0, 0)),
            pl.BlockSpec((HID, HID), lambda i: (0, 0)),
            pl.BlockSpec((HID, HID), lambda i: (0, 0)),
            pl.BlockSpec((1, HID), lambda i: (0, 0)),
            pl.BlockSpec((HID, HID // 2), lambda i: (0, 0)),
            pl.BlockSpec((1, HID // 2), lambda i: (0, 0)),
            pl.BlockSpec((HID // 2, OUT), lambda i: (0, 0)),
            pl.BlockSpec((1, OUT), lambda i: (0, 0)),
            pl.BlockSpec((HID, OUT), lambda i: (0, 0)),
            pl.BlockSpec((1, OUT), lambda i: (0, 0)),
        ],
        out_specs=[
            pl.BlockSpec((1, OUT), lambda i: (0, 0)),
            pl.BlockSpec((1, OUT), lambda i: (0, 0)),
        ],
        out_shape=[
            jax.ShapeDtypeStruct((1, OUT), jnp.float32),
            jax.ShapeDtypeStruct((1, OUT), jnp.float32),
        ],
        scratch_shapes=[pltpu.VMEM((1, HID), jnp.float32)],
    )(xlg, msg_parts, xh, eself, esum4, b_gat, wpa, wpb, b_pool,
      W_c1, b_c1, W_c2, b_c2, W_u, b_u)


def kernel(x, local_edge_index, global_edge_index, W_lp, b_lp, W_gcn, b_gcn,
           W_gat, att_src, att_dst, b_gat, W_pool, b_pool, W_c1, b_c1,
           W_c2, b_c2, W_u, b_u):
    row_l = local_edge_index[0]
    col_l = local_edge_index[1]
    row_g = global_edge_index[0]
    col_g = global_edge_index[1]

    deg_part = jnp.zeros((N, NW), jnp.float32).at[col_l, 0].add(1.0)

    asrT = att_src.reshape(H, HID)
    adrT = att_dst.reshape(H, HID)
    xwd, xh, asrc16, adst16, eself = _tc1(
        x, deg_part, W_lp, b_lp.reshape(1, HID), W_gcn, W_gat, asrT, adrT)

    acc0 = jnp.zeros((N, HID), jnp.float32).at[col_l].add(xwd[row_l])
    acc_parts = jnp.stack([acc0, jnp.zeros((N, HID), jnp.float32)])

    ea = jnp.exp(_lrelu(asrc16[row_g, :H] + adst16[col_g, :H]))
    esum_parts = jnp.zeros((NW, N, H), jnp.float32).at[0].add(
        jnp.zeros((N, H), jnp.float32).at[col_g].add(ea))

    xlg, esum16, esum4 = _tc2(
        acc_parts, xwd, deg_part, b_gcn.reshape(1, HID), esum_parts, eself)

    att = ea / esum4[col_g]
    msg = jnp.zeros((N, HID), jnp.float32)
    for h in range(H):
        msg = msg.at[col_g].add(att[:, h:h + 1] * xh[row_g, h * HID:(h + 1) * HID])
    msg_parts = jnp.stack([msg, jnp.zeros((N, HID), jnp.float32)])

    pred, unc = _tc3(xlg, msg_parts, xh, eself, esum4,
                     b_gat.reshape(1, HID), W_pool, b_pool.reshape(1, HID),
                     W_c1, b_c1.reshape(1, HID // 2), W_c2,
                     b_c2.reshape(1, OUT), W_u, b_u.reshape(1, OUT))
    return (pred, unc)


# speedup vs baseline: 147.9596x; 147.9596x over previous
import functools
import jax
import jax.numpy as jnp
from jax import lax
from jax.experimental import pallas as pl
from jax.experimental.pallas import tpu as pltpu
from jax.experimental.pallas import tpu_sc as plsc

N = 10000
D = 128
HID = 128
OUT = 16
EL = 320000
EG = 80000
H = 4

NC = 2
NS = 16
NW = NC * NS

BN = 1000
NGRID = N // BN
EGP = 81920
EG_PT = EGP // NW

PREC = lax.Precision.HIGHEST


def _dot(a, b):
    return jnp.dot(a, b, precision=PREC)


def _lrelu(v):
    return jnp.where(v > 0, v, 0.2 * v)


def _tc1_body(x_ref, degp_ref, wlp_ref, blp_ref, wgcn_ref, wgat_ref,
              asr_ref, adr_ref,
              xwd_o, xh_o, asrc16_o, adst16_o, eself_o):
    xb = x_ref[...]
    xl = jnp.maximum(_dot(xb, wlp_ref[...]) + blp_ref[...], 0.0)
    deg = jnp.sum(degp_ref[...], axis=1) + 1.0
    dis = lax.rsqrt(deg)
    xw = _dot(xl, wgcn_ref[...])
    xwd_o[...] = dis[:, None] * xw
    xh = _dot(xl, wgat_ref[...])
    xh_o[...] = xh
    acols = []
    bcols = []
    for h in range(H):
        xh_h = xh[:, h * HID:(h + 1) * HID]
        acols.append(_dot(xh_h, asr_ref[...][h][:, None]))
        bcols.append(_dot(xh_h, adr_ref[...][h][:, None]))
    a_src = jnp.concatenate(acols, axis=1)
    a_dst = jnp.concatenate(bcols, axis=1)
    zpad = jnp.zeros((a_src.shape[0], 16 - H), jnp.float32)
    asrc16_o[...] = jnp.concatenate([a_src, zpad], axis=1)
    adst16_o[...] = jnp.concatenate([a_dst, zpad], axis=1)
    eself_o[...] = jnp.exp(_lrelu(a_src + a_dst))


def _tc1(x, deg_part, W_lp, b_lp, W_gcn, W_gat, asrT, adrT):
    return pl.pallas_call(
        _tc1_body,
        grid=(NGRID,),
        in_specs=[
            pl.BlockSpec((BN, D), lambda i: (i, 0)),
            pl.BlockSpec((BN, NW), lambda i: (i, 0)),
            pl.BlockSpec((D, HID), lambda i: (0, 0)),
            pl.BlockSpec((1, HID), lambda i: (0, 0)),
            pl.BlockSpec((HID, HID), lambda i: (0, 0)),
            pl.BlockSpec((HID, H * HID), lambda i: (0, 0)),
            pl.BlockSpec((H, HID), lambda i: (0, 0)),
            pl.BlockSpec((H, HID), lambda i: (0, 0)),
        ],
        out_specs=[
            pl.BlockSpec((BN, HID), lambda i: (i, 0)),
            pl.BlockSpec((BN, H * HID), lambda i: (i, 0)),
            pl.BlockSpec((BN, 16), lambda i: (i, 0)),
            pl.BlockSpec((BN, 16), lambda i: (i, 0)),
            pl.BlockSpec((BN, H), lambda i: (i, 0)),
        ],
        out_shape=[
            jax.ShapeDtypeStruct((N, HID), jnp.float32),
            jax.ShapeDtypeStruct((N, H * HID), jnp.float32),
            jax.ShapeDtypeStruct((N, 16), jnp.float32),
            jax.ShapeDtypeStruct((N, 16), jnp.float32),
            jax.ShapeDtypeStruct((N, H), jnp.float32),
        ],
    )(x, deg_part, W_lp, b_lp, W_gcn, W_gat, asrT, adrT)


def _tc2_body(accp_ref, xwd_ref, degp_ref, bgcn_ref, esump_ref, eself_ref,
              xlg_o, esum16_o, esum4_o):
    deg = jnp.sum(degp_ref[...], axis=1) + 1.0
    dis = lax.rsqrt(deg)
    acc = accp_ref[...][0] + accp_ref[...][1] + xwd_ref[...]
    xlg_o[...] = jnp.maximum(dis[:, None] * acc + bgcn_ref[...], 0.0)
    esum = jnp.sum(esump_ref[...], axis=0) + eself_ref[...]
    esum4_o[...] = esum
    opad = jnp.ones((esum.shape[0], 16 - H), jnp.float32)
    esum16_o[...] = jnp.concatenate([esum, opad], axis=1)


def _tc2(acc_parts, xwd, deg_part, b_gcn, esum_parts, eself):
    return pl.pallas_call(
        _tc2_body,
        grid=(NGRID,),
        in_specs=[
            pl.BlockSpec((NC, BN, HID), lambda i: (0, i, 0)),
            pl.BlockSpec((BN, HID), lambda i: (i, 0)),
            pl.BlockSpec((BN, NW), lambda i: (i, 0)),
            pl.BlockSpec((1, HID), lambda i: (0, 0)),
            pl.BlockSpec((NW, BN, H), lambda i: (0, i, 0)),
            pl.BlockSpec((BN, H), lambda i: (i, 0)),
        ],
        out_specs=[
            pl.BlockSpec((BN, HID), lambda i: (i, 0)),
            pl.BlockSpec((BN, 16), lambda i: (i, 0)),
            pl.BlockSpec((BN, H), lambda i: (i, 0)),
        ],
        out_shape=[
            jax.ShapeDtypeStruct((N, HID), jnp.float32),
            jax.ShapeDtypeStruct((N, 16), jnp.float32),
            jax.ShapeDtypeStruct((N, H), jnp.float32),
        ],
    )(acc_parts, xwd, deg_part, b_gcn, esum_parts, eself)


def _tc3_body(xlg_ref, msgp_ref, xh_ref, eself_ref, esum4_ref, bgat_ref,
              wpa_ref, wpb_ref, bpool_ref, wc1_ref, bc1_ref, wc2_ref,
              bc2_ref, wu_ref, bu_ref,
              pred_o, unc_o, acc_s):
    i = pl.program_id(0)
    att_self = eself_ref[...] / esum4_ref[...]
    xh = xh_ref[...]
    self_msg = att_self[:, 0:1] * xh[:, 0:HID]
    for h in range(1, H):
        self_msg = self_msg + att_self[:, h:h + 1] * xh[:, h * HID:(h + 1) * HID]
    gat = jnp.maximum(
        (msgp_ref[...][0] + msgp_ref[...][1] + self_msg) * (1.0 / H)
        + bgat_ref[...], 0.0)
    pooled = jnp.maximum(
        _dot(xlg_ref[...], wpa_ref[...]) + _dot(gat, wpb_ref[...])
        + bpool_ref[...], 0.0)
    psum = jnp.sum(pooled, axis=0, keepdims=True)

    @pl.when(i == 0)
    def _():
        acc_s[...] = psum

    @pl.when(i > 0)
    def _():
        acc_s[...] = acc_s[...] + psum

    @pl.when(i == NGRID - 1)
    def _():
        xf = acc_s[...] * (1.0 / N)
        hh = jnp.maximum(_dot(xf, wc1_ref[...]) + bc1_ref[...], 0.0)
        pred_o[...] = _dot(hh, wc2_ref[...]) + bc2_ref[...]
        zu = _dot(xf, wu_ref[...]) + bu_ref[...]
        unc_o[...] = 1.0 / (1.0 + jnp.exp(-zu))


def _tc3(xlg, msg_parts, xh, eself, esum4, b_gat, W_pool, b_pool,
         W_c1, b_c1, W_c2, b_c2, W_u, b_u):
    wpa = W_pool[:HID]
    wpb = W_pool[HID:]
    return pl.pallas_call(
        _tc3_body,
        grid=(NGRID,),
        in_specs=[
            pl.BlockSpec((BN, HID), lambda i: (i, 0)),
            pl.BlockSpec((NC, BN, HID), lambda i: (0, i, 0)),
            pl.BlockSpec((BN, H * HID), lambda i: (i, 0)),
            pl.BlockSpec((BN, H), lambda i: (i, 0)),
            pl.BlockSpec((BN, H), lambda i: (i, 0)),
            pl.BlockSpec((1, HID), lambda i: (0, 0)),
            pl.BlockSpec((HID, HID), lambda i: (0, 0)),
            pl.BlockSpec((HID, HID), lambda i: (0, 0)),
            pl.BlockSpec((1, HID), lambda i: (0, 0)),
            pl.BlockSpec((HID, HID // 2), lambda i: (0, 0)),
            pl.BlockSpec((1, HID // 2), lambda i: (0, 0)),
            pl.BlockSpec((HID // 2, OUT), lambda i: (0, 0)),
            pl.BlockSpec((1, OUT), lambda i: (0, 0)),
            pl.BlockSpec((HID, OUT), lambda i: (0, 0)),
            pl.BlockSpec((1, OUT), lambda i: (0, 0)),
        ],
        out_specs=[
            pl.BlockSpec((1, OUT), lambda i: (0, 0)),
            pl.BlockSpec((1, OUT), lambda i: (0, 0)),
        ],
        out_shape=[
            jax.ShapeDtypeStruct((1, OUT), jnp.float32),
            jax.ShapeDtypeStruct((1, OUT), jnp.float32),
        ],
        scratch_shapes=[pltpu.VMEM((1, HID), jnp.float32)],
    )(xlg, msg_parts, xh, eself, esum4, b_gat, wpa, wpb, b_pool,
      W_c1, b_c1, W_c2, b_c2, W_u, b_u)


EL_PT = EL // NW
KL = 200
CHL = EL_PT // KL
KG = 128
CHG = EG_PT // KG
KG2 = 32
CHG2 = EG_PT // KG2

_SC_MESH = plsc.VectorSubcoreMesh(
    core_axis_name="c", subcore_axis_name="s",
    num_cores=NC, num_subcores=NS)
_SC_PARAMS = pltpu.CompilerParams(needs_layout_passes=False, use_tc_tiling_on_sc=False)


def _wid():
    return lax.axis_index("s") * NC + lax.axis_index("c")


def _sc_deg(col2):
    @functools.partial(
        pl.kernel,
        out_type=jax.ShapeDtypeStruct((NW, N), jnp.float32),
        mesh=_SC_MESH,
        compiler_params=_SC_PARAMS,
        scratch_types=[
            pltpu.VMEM((EL_PT,), jnp.int32),
            pltpu.VMEM((N,), jnp.float32),
        ],
    )
    def body(col_hbm, out_hbm, col_v, deg_v):
        wid = _wid()
        pltpu.sync_copy(col_hbm.at[wid], col_v)
        z16 = jnp.zeros((16,), jnp.float32)
        o16 = jnp.ones((16,), jnp.float32)

        def zb(i, c):
            deg_v[pl.ds(i * 16, 16)] = z16
            return c
        lax.fori_loop(0, N // 16, zb, 0)

        def eb(i, c):
            idx = col_v[pl.ds(i * 16, 16)]
            plsc.addupdate_scatter(deg_v, [idx], o16)
            return c
        lax.fori_loop(0, EL_PT // 16, eb, 0)
        pltpu.sync_copy(deg_v, out_hbm.at[wid])

    return body(col2)


def _sc_gcn(xwd, row3, col3, zinit):
    npt = N // NS

    @functools.partial(
        pl.kernel,
        out_type=jax.ShapeDtypeStruct((NC, N, HID), jnp.float32),
        mesh=_SC_MESH,
        compiler_params=_SC_PARAMS,
        scratch_types=[
            pltpu.VMEM((KL,), jnp.int32),
            pltpu.VMEM((KL,), jnp.int32),
            pltpu.VMEM((KL, HID), jnp.float32),
            pltpu.VMEM_SHARED((N, HID), jnp.float32),
            pltpu.SemaphoreType.DMA,
        ],
    )
    def body(xwd_hbm, row_hbm, col_hbm, zinit_hbm, out_hbm,
             row_v, col_v, rows_v, acc_sh, sem):
        cid = lax.axis_index("c")
        sid = lax.axis_index("s")
        wid = _wid()
        pltpu.sync_copy(zinit_hbm.at[pl.ds(sid * npt, npt)],
                        acc_sh.at[pl.ds(sid * npt, npt)])
        plsc.subcore_barrier()

        def chunk(k, c):
            pltpu.sync_copy(row_hbm.at[wid, k], row_v)
            pltpu.sync_copy(col_hbm.at[wid, k], col_v)
            pltpu.async_copy(xwd_hbm.at[row_v], rows_v, sem).wait()
            pltpu.sync_copy(rows_v, acc_sh.at[col_v], add=True)
            return c
        lax.fori_loop(0, CHL, chunk, 0)
        plsc.subcore_barrier()
        pltpu.sync_copy(acc_sh.at[pl.ds(sid * npt, npt)],
                        out_hbm.at[cid, pl.ds(sid * npt, npt)])

    return body(xwd, row3, col3, zinit)


def _sc_gat1(asrc16, adst16, row3, col3):
    @functools.partial(
        pl.kernel,
        out_type=[
            jax.ShapeDtypeStruct((EGP, 16), jnp.float32),
            jax.ShapeDtypeStruct((NW, N * H), jnp.float32),
        ],
        mesh=_SC_MESH,
        compiler_params=_SC_PARAMS,
        scratch_types=[
            pltpu.VMEM((KG,), jnp.int32),
            pltpu.VMEM((KG,), jnp.int32),
            pltpu.VMEM((KG, 16), jnp.float32),
            pltpu.VMEM((KG, 16), jnp.float32),
            pltpu.VMEM((KG, 16), jnp.float32),
            pltpu.VMEM((N * H,), jnp.float32),
            pltpu.SemaphoreType.DMA,
        ],
    )
    def body(asrc_hbm, adst_hbm, row_hbm, col_hbm, ea_hbm, esum_hbm,
             row_v, col_v, va, vb, ea_v, esum_v, sem):
        wid = _wid()
        z16 = jnp.zeros((16,), jnp.float32)
        lane = lax.iota(jnp.int32, 16)

        def zb(i, c):
            esum_v[pl.ds(i * 16, 16)] = z16
            return c
        lax.fori_loop(0, (N * H) // 16, zb, 0)

        def chunk(k, c):
            pltpu.sync_copy(row_hbm.at[wid, k], row_v)
            pltpu.sync_copy(col_hbm.at[wid, k], col_v)
            pltpu.async_copy(asrc_hbm.at[row_v], va, sem).wait()
            pltpu.async_copy(adst_hbm.at[col_v], vb, sem).wait()
            base = wid * EG_PT + k * KG

            def gb(g, c2):
                cv = col_v[pl.ds(g * 16, 16)]
                for t in range(16):
                    e = g * 16 + t
                    alpha = va[e] + vb[e]
                    alpha = jnp.where(alpha > 0, alpha, 0.2 * alpha)
                    eav = jnp.exp(alpha)
                    valid = (base + e) < EG
                    keep = jnp.logical_and(lane < H, valid)
                    eav = jnp.where(keep, eav, 0.0)
                    ea_v[e] = eav
                    plsc.addupdate_scatter(esum_v, [cv[t] * H + lane], eav,
                                           mask=lane < H)
                return c2
            lax.fori_loop(0, KG // 16, gb, 0)
            pltpu.sync_copy(ea_v, ea_hbm.at[pl.ds(base, KG)])
            return c
        lax.fori_loop(0, CHG, chunk, 0)
        pltpu.sync_copy(esum_v, esum_hbm.at[wid])

    return body(asrc16, adst16, row3, col3)


def _sc_gat2(xh, ea16, esum16, row3, col3, zinit):
    npt = N // NS

    @functools.partial(
        pl.kernel,
        out_type=jax.ShapeDtypeStruct((NC, N, HID), jnp.float32),
        mesh=_SC_MESH,
        compiler_params=_SC_PARAMS,
        scratch_types=[
            pltpu.VMEM((KG2,), jnp.int32),
            pltpu.VMEM((KG2,), jnp.int32),
            pltpu.VMEM((KG2, H * HID), jnp.float32),
            pltpu.VMEM((KG2, 16), jnp.float32),
            pltpu.VMEM((KG2, 16), jnp.float32),
            pltpu.VMEM((KG2, HID), jnp.float32),
            pltpu.VMEM_SHARED((N, HID), jnp.float32),
            pltpu.SemaphoreType.DMA,
        ],
    )
    def body(xh_hbm, ea_hbm, esum_hbm, row_hbm, col_hbm, zinit_hbm, out_hbm,
             row_v, col_v, xh_v, ea_v, es_v, msg_v, acc_sh, sem):
        cid = lax.axis_index("c")
        sid = lax.axis_index("s")
        wid = _wid()
        pltpu.sync_copy(zinit_hbm.at[pl.ds(sid * npt, npt)],
                        acc_sh.at[pl.ds(sid * npt, npt)])
        plsc.subcore_barrier()

        def chunk(k, c):
            base = wid * EG_PT + k * KG2
            pltpu.sync_copy(row_hbm.at[wid, k], row_v)
            pltpu.sync_copy(col_hbm.at[wid, k], col_v)
            pltpu.sync_copy(ea_hbm.at[pl.ds(base, KG2)], ea_v)
            pltpu.async_copy(xh_hbm.at[row_v], xh_v, sem).wait()
            pltpu.async_copy(esum_hbm.at[col_v], es_v, sem).wait()

            def eb(e, c2):
                att = ea_v[e] / es_v[e]
                m = [None] * (HID // 16)
                for h in range(H):
                    ah = jnp.full((16,), att[h], jnp.float32)
                    for j in range(HID // 16):
                        xv = xh_v[e, pl.ds(h * HID + j * 16, 16)]
                        m[j] = ah * xv if h == 0 else m[j] + ah * xv
                for j in range(HID // 16):
                    msg_v[e, pl.ds(j * 16, 16)] = m[j]
                return c2
            lax.fori_loop(0, KG2, eb, 0)
            pltpu.sync_copy(msg_v, acc_sh.at[col_v], add=True)
            return c
        lax.fori_loop(0, CHG2, chunk, 0)
        plsc.subcore_barrier()
        pltpu.sync_copy(acc_sh.at[pl.ds(sid * npt, npt)],
                        out_hbm.at[cid, pl.ds(sid * npt, npt)])

    return body(xh, ea16, esum16, row3, col3, zinit)


def kernel(x, local_edge_index, global_edge_index, W_lp, b_lp, W_gcn, b_gcn,
           W_gat, att_src, att_dst, b_gat, W_pool, b_pool, W_c1, b_c1,
           W_c2, b_c2, W_u, b_u):
    row_l = local_edge_index[0]
    col_l = local_edge_index[1]
    pad = jnp.zeros((EGP - EG,), jnp.int32)
    row_g3 = jnp.concatenate([global_edge_index[0], pad]).reshape(NW, -1)
    col_g3 = jnp.concatenate([global_edge_index[1], pad]).reshape(NW, -1)
    zinit = jnp.zeros((N, HID), jnp.float32)

    deg_parts = _sc_deg(col_l.reshape(NW, EL_PT))
    deg_part = deg_parts.T

    asrT = att_src.reshape(H, HID)
    adrT = att_dst.reshape(H, HID)
    xwd, xh, asrc16, adst16, eself = _tc1(
        x, deg_part, W_lp, b_lp.reshape(1, HID), W_gcn, W_gat, asrT, adrT)

    acc_parts = _sc_gcn(xwd, row_l.reshape(NW, CHL, KL),
                        col_l.reshape(NW, CHL, KL), zinit)

    ea16, esum_pf = _sc_gat1(asrc16, adst16,
                             row_g3.reshape(NW, CHG, KG),
                             col_g3.reshape(NW, CHG, KG))
    esum_parts = esum_pf.reshape(NW, N, H)

    xlg, esum16, esum4 = _tc2(
        acc_parts, xwd, deg_part, b_gcn.reshape(1, HID), esum_parts, eself)

    msg_parts = _sc_gat2(xh, ea16, esum16,
                         row_g3.reshape(NW, CHG2, KG2),
                         col_g3.reshape(NW, CHG2, KG2), zinit)

    pred, unc = _tc3(xlg, msg_parts, xh, eself, esum4,
                     b_gat.reshape(1, HID), W_pool, b_pool.reshape(1, HID),
                     W_c1, b_c1.reshape(1, HID // 2), W_c2,
                     b_c2.reshape(1, OUT), W_u, b_u.reshape(1, OUT))
    return (pred, unc)


# speedup vs baseline: 221.2286x; 1.4952x over previous
import functools
import jax
import jax.numpy as jnp
from jax import lax
from jax.experimental import pallas as pl
from jax.experimental.pallas import tpu as pltpu
from jax.experimental.pallas import tpu_sc as plsc

N = 10000
D = 128
HID = 128
OUT = 16
EL = 320000
EG = 80000
H = 4

NC = 2
NS = 16
NW = NC * NS

BN = 1000
NGRID = N // BN
EGP = 81920
EG_PT = EGP // NW

PREC = lax.Precision.HIGHEST


def _dot(a, b):
    return jnp.dot(a, b, precision=PREC)


def _lrelu(v):
    return jnp.where(v > 0, v, 0.2 * v)


def _tc1_body(x_ref, degp_ref, wlp_ref, blp_ref, wgcn_ref, wgat_ref,
              asr_ref, adr_ref,
              xwd_o, xh_o, asrc16_o, adst16_o, eself_o):
    xb = x_ref[...]
    xl = jnp.maximum(_dot(xb, wlp_ref[...]) + blp_ref[...], 0.0)
    deg = jnp.sum(degp_ref[...], axis=1) + 1.0
    dis = lax.rsqrt(deg)
    xw = _dot(xl, wgcn_ref[...])
    xwd_o[...] = dis[:, None] * xw
    xh = _dot(xl, wgat_ref[...])
    xh_o[...] = xh
    acols = []
    bcols = []
    for h in range(H):
        xh_h = xh[:, h * HID:(h + 1) * HID]
        acols.append(_dot(xh_h, asr_ref[...][h][:, None]))
        bcols.append(_dot(xh_h, adr_ref[...][h][:, None]))
    a_src = jnp.concatenate(acols, axis=1)
    a_dst = jnp.concatenate(bcols, axis=1)
    zpad = jnp.zeros((a_src.shape[0], 16 - H), jnp.float32)
    asrc16_o[...] = jnp.concatenate([a_src, zpad], axis=1)
    adst16_o[...] = jnp.concatenate([a_dst, zpad], axis=1)
    eself_o[...] = jnp.exp(_lrelu(a_src + a_dst))


def _tc1(x, deg_part, W_lp, b_lp, W_gcn, W_gat, asrT, adrT):
    return pl.pallas_call(
        _tc1_body,
        grid=(NGRID,),
        in_specs=[
            pl.BlockSpec((BN, D), lambda i: (i, 0)),
            pl.BlockSpec((BN, NW), lambda i: (i, 0)),
            pl.BlockSpec((D, HID), lambda i: (0, 0)),
            pl.BlockSpec((1, HID), lambda i: (0, 0)),
            pl.BlockSpec((HID, HID), lambda i: (0, 0)),
            pl.BlockSpec((HID, H * HID), lambda i: (0, 0)),
            pl.BlockSpec((H, HID), lambda i: (0, 0)),
            pl.BlockSpec((H, HID), lambda i: (0, 0)),
        ],
        out_specs=[
            pl.BlockSpec((BN, HID), lambda i: (i, 0)),
            pl.BlockSpec((BN, H * HID), lambda i: (i, 0)),
            pl.BlockSpec((BN, 16), lambda i: (i, 0)),
            pl.BlockSpec((BN, 16), lambda i: (i, 0)),
            pl.BlockSpec((BN, H), lambda i: (i, 0)),
        ],
        out_shape=[
            jax.ShapeDtypeStruct((N, HID), jnp.float32),
            jax.ShapeDtypeStruct((N, H * HID), jnp.float32),
            jax.ShapeDtypeStruct((N, 16), jnp.float32),
            jax.ShapeDtypeStruct((N, 16), jnp.float32),
            jax.ShapeDtypeStruct((N, H), jnp.float32),
        ],
    )(x, deg_part, W_lp, b_lp, W_gcn, W_gat, asrT, adrT)


def _tc2_body(accp_ref, xwd_ref, degp_ref, bgcn_ref, esump_ref, eself_ref,
              xlg_o, esum16_o, esum4_o):
    deg = jnp.sum(degp_ref[...], axis=1) + 1.0
    dis = lax.rsqrt(deg)
    acc = accp_ref[...][0] + accp_ref[...][1] + xwd_ref[...]
    xlg_o[...] = jnp.maximum(dis[:, None] * acc + bgcn_ref[...], 0.0)
    esum = jnp.sum(esump_ref[...], axis=0) + eself_ref[...]
    esum4_o[...] = esum
    opad = jnp.ones((esum.shape[0], 16 - H), jnp.float32)
    esum16_o[...] = jnp.concatenate([esum, opad], axis=1)


def _tc2(acc_parts, xwd, deg_part, b_gcn, esum_parts, eself):
    return pl.pallas_call(
        _tc2_body,
        grid=(NGRID,),
        in_specs=[
            pl.BlockSpec((NC, BN, HID), lambda i: (0, i, 0)),
            pl.BlockSpec((BN, HID), lambda i: (i, 0)),
            pl.BlockSpec((BN, NW), lambda i: (i, 0)),
            pl.BlockSpec((1, HID), lambda i: (0, 0)),
            pl.BlockSpec((NW, BN, H), lambda i: (0, i, 0)),
            pl.BlockSpec((BN, H), lambda i: (i, 0)),
        ],
        out_specs=[
            pl.BlockSpec((BN, HID), lambda i: (i, 0)),
            pl.BlockSpec((BN, 16), lambda i: (i, 0)),
            pl.BlockSpec((BN, H), lambda i: (i, 0)),
        ],
        out_shape=[
            jax.ShapeDtypeStruct((N, HID), jnp.float32),
            jax.ShapeDtypeStruct((N, 16), jnp.float32),
            jax.ShapeDtypeStruct((N, H), jnp.float32),
        ],
    )(acc_parts, xwd, deg_part, b_gcn, esum_parts, eself)


def _tc3_body(xlg_ref, msgp_ref, xh_ref, eself_ref, esum4_ref, bgat_ref,
              wpa_ref, wpb_ref, bpool_ref, wc1_ref, bc1_ref, wc2_ref,
              bc2_ref, wu_ref, bu_ref,
              pred_o, unc_o, acc_s):
    i = pl.program_id(0)
    att_self = eself_ref[...] / esum4_ref[...]
    xh = xh_ref[...]
    self_msg = att_self[:, 0:1] * xh[:, 0:HID]
    for h in range(1, H):
        self_msg = self_msg + att_self[:, h:h + 1] * xh[:, h * HID:(h + 1) * HID]
    gat = jnp.maximum(
        (msgp_ref[...][0] + msgp_ref[...][1] + self_msg) * (1.0 / H)
        + bgat_ref[...], 0.0)
    pooled = jnp.maximum(
        _dot(xlg_ref[...], wpa_ref[...]) + _dot(gat, wpb_ref[...])
        + bpool_ref[...], 0.0)
    psum = jnp.sum(pooled, axis=0, keepdims=True)

    @pl.when(i == 0)
    def _():
        acc_s[...] = psum

    @pl.when(i > 0)
    def _():
        acc_s[...] = acc_s[...] + psum

    @pl.when(i == NGRID - 1)
    def _():
        xf = acc_s[...] * (1.0 / N)
        hh = jnp.maximum(_dot(xf, wc1_ref[...]) + bc1_ref[...], 0.0)
        pred_o[...] = _dot(hh, wc2_ref[...]) + bc2_ref[...]
        zu = _dot(xf, wu_ref[...]) + bu_ref[...]
        unc_o[...] = 1.0 / (1.0 + jnp.exp(-zu))


def _tc3(xlg, msg_parts, xh, eself, esum4, b_gat, W_pool, b_pool,
         W_c1, b_c1, W_c2, b_c2, W_u, b_u):
    wpa = W_pool[:HID]
    wpb = W_pool[HID:]
    return pl.pallas_call(
        _tc3_body,
        grid=(NGRID,),
        in_specs=[
            pl.BlockSpec((BN, HID), lambda i: (i, 0)),
            pl.BlockSpec((NC, BN, HID), lambda i: (0, i, 0)),
            pl.BlockSpec((BN, H * HID), lambda i: (i, 0)),
            pl.BlockSpec((BN, H), lambda i: (i, 0)),
            pl.BlockSpec((BN, H), lambda i: (i, 0)),
            pl.BlockSpec((1, HID), lambda i: (0, 0)),
            pl.BlockSpec((HID, HID), lambda i: (0, 0)),
            pl.BlockSpec((HID, HID), lambda i: (0, 0)),
            pl.BlockSpec((1, HID), lambda i: (0, 0)),
            pl.BlockSpec((HID, HID // 2), lambda i: (0, 0)),
            pl.BlockSpec((1, HID // 2), lambda i: (0, 0)),
            pl.BlockSpec((HID // 2, OUT), lambda i: (0, 0)),
            pl.BlockSpec((1, OUT), lambda i: (0, 0)),
            pl.BlockSpec((HID, OUT), lambda i: (0, 0)),
            pl.BlockSpec((1, OUT), lambda i: (0, 0)),
        ],
        out_specs=[
            pl.BlockSpec((1, OUT), lambda i: (0, 0)),
            pl.BlockSpec((1, OUT), lambda i: (0, 0)),
        ],
        out_shape=[
            jax.ShapeDtypeStruct((1, OUT), jnp.float32),
            jax.ShapeDtypeStruct((1, OUT), jnp.float32),
        ],
        scratch_shapes=[pltpu.VMEM((1, HID), jnp.float32)],
    )(xlg, msg_parts, xh, eself, esum4, b_gat, wpa, wpb, b_pool,
      W_c1, b_c1, W_c2, b_c2, W_u, b_u)


EL_PT = EL // NW
KL = 80
CHL = EL_PT // KL
KG = 640
CHG = EG_PT // KG
KG2 = 32
CHG2 = EG_PT // KG2

_SC_MESH = plsc.VectorSubcoreMesh(
    core_axis_name="c", subcore_axis_name="s",
    num_cores=NC, num_subcores=NS)
_SC_PARAMS = pltpu.CompilerParams(needs_layout_passes=False, use_tc_tiling_on_sc=False)


def _wid():
    return lax.axis_index("s") * NC + lax.axis_index("c")


def _sc_deg(col2):
    @functools.partial(
        pl.kernel,
        out_type=jax.ShapeDtypeStruct((NW, N), jnp.float32),
        mesh=_SC_MESH,
        compiler_params=_SC_PARAMS,
        scratch_types=[
            pltpu.VMEM((EL_PT,), jnp.int32),
            pltpu.VMEM((N,), jnp.float32),
        ],
    )
    def body(col_hbm, out_hbm, col_v, deg_v):
        wid = _wid()
        pltpu.sync_copy(col_hbm.at[wid], col_v)
        z16 = jnp.zeros((16,), jnp.float32)
        o16 = jnp.ones((16,), jnp.float32)

        def zb(i, c):
            deg_v[pl.ds(i * 16, 16)] = z16
            return c
        lax.fori_loop(0, N // 16, zb, 0)

        def eb(i, c):
            idx = col_v[pl.ds(i * 16, 16)]
            plsc.addupdate_scatter(deg_v, [idx], o16)
            return c
        lax.fori_loop(0, EL_PT // 16, eb, 0)
        pltpu.sync_copy(deg_v, out_hbm.at[wid])

    return body(col2)


def _sc_gcn(xwd, row3, col3, zinit):
    npt = N // NS

    @functools.partial(
        pl.kernel,
        out_type=jax.ShapeDtypeStruct((NC, N, HID), jnp.float32),
        mesh=_SC_MESH,
        compiler_params=_SC_PARAMS,
        scratch_types=[
            pltpu.VMEM((CHL, KL), jnp.int32),
            pltpu.VMEM((CHL, KL), jnp.int32),
            pltpu.VMEM((2, KL, HID), jnp.float32),
            pltpu.VMEM_SHARED((N, HID), jnp.float32),
            pltpu.SemaphoreType.DMA,
        ],
    )
    def body(xwd_hbm, row_hbm, col_hbm, zinit_hbm, out_hbm,
             row_v, col_v, rows_v, acc_sh, sem):
        cid = lax.axis_index("c")
        sid = lax.axis_index("s")
        wid = _wid()
        pltpu.sync_copy(row_hbm.at[wid], row_v)
        pltpu.sync_copy(col_hbm.at[wid], col_v)
        pltpu.sync_copy(zinit_hbm.at[pl.ds(sid * npt, npt)],
                        acc_sh.at[pl.ds(sid * npt, npt)])
        plsc.subcore_barrier()
        pltpu.async_copy(xwd_hbm.at[row_v.at[0]], rows_v.at[0], sem)
        pltpu.async_copy(xwd_hbm.at[row_v.at[1]], rows_v.at[1], sem)

        def pair(i, c):
            for b in range(2):
                k = 2 * i + b
                pltpu.make_async_copy(
                    xwd_hbm.at[pl.ds(0, KL)], rows_v.at[b], sem).wait()
                pltpu.sync_copy(rows_v.at[b], acc_sh.at[col_v.at[k]], add=True)

                @pl.when(k + 2 < CHL)
                def _():
                    pltpu.async_copy(xwd_hbm.at[row_v.at[k + 2]],
                                     rows_v.at[b], sem)
            return c
        lax.fori_loop(0, CHL // 2, pair, 0)
        if CHL % 2:
            k = CHL - 1
            b = k % 2
            pltpu.make_async_copy(
                xwd_hbm.at[pl.ds(0, KL)], rows_v.at[b], sem).wait()
            pltpu.sync_copy(rows_v.at[b], acc_sh.at[col_v.at[k]], add=True)
        plsc.subcore_barrier()
        pltpu.sync_copy(acc_sh.at[pl.ds(sid * npt, npt)],
                        out_hbm.at[cid, pl.ds(sid * npt, npt)])

    return body(xwd, row3, col3, zinit)


def _sc_gat1(asrc16, adst16, row3, col3):
    @functools.partial(
        pl.kernel,
        out_type=[
            jax.ShapeDtypeStruct((EGP, 16), jnp.float32),
            jax.ShapeDtypeStruct((NW, N * H), jnp.float32),
        ],
        mesh=_SC_MESH,
        compiler_params=_SC_PARAMS,
        scratch_types=[
            pltpu.VMEM((CHG, KG), jnp.int32),
            pltpu.VMEM((CHG, KG), jnp.int32),
            pltpu.VMEM((KG, 16), jnp.float32),
            pltpu.VMEM((KG, 16), jnp.float32),
            pltpu.VMEM((KG, 16), jnp.float32),
            pltpu.VMEM((N * H,), jnp.float32),
            pltpu.SemaphoreType.DMA,
        ],
    )
    def body(asrc_hbm, adst_hbm, row_hbm, col_hbm, ea_hbm, esum_hbm,
             row_v, col_v, va, vb, ea_v, esum_v, sem):
        wid = _wid()
        z16 = jnp.zeros((16,), jnp.float32)
        lane = lax.iota(jnp.int32, 16)
        pltpu.sync_copy(row_hbm.at[wid], row_v)
        pltpu.sync_copy(col_hbm.at[wid], col_v)

        def zb(i, c):
            esum_v[pl.ds(i * 16, 16)] = z16
            return c
        lax.fori_loop(0, (N * H) // 16, zb, 0)

        def chunk(k, c):
            pltpu.async_copy(asrc_hbm.at[row_v.at[k]], va, sem)
            pltpu.async_copy(adst_hbm.at[col_v.at[k]], vb, sem)
            pltpu.make_async_copy(asrc_hbm.at[pl.ds(0, KG)], va, sem).wait()
            pltpu.make_async_copy(asrc_hbm.at[pl.ds(0, KG)], vb, sem).wait()
            base = wid * EG_PT + k * KG

            def gb(g, c2):
                cv = col_v[k, pl.ds(g * 16, 16)]
                for t in range(16):
                    e = g * 16 + t
                    alpha = va[e] + vb[e]
                    alpha = jnp.where(alpha > 0, alpha, 0.2 * alpha)
                    eav = jnp.exp(alpha)
                    valid = (base + e) < EG
                    keep = jnp.logical_and(lane < H, valid)
                    eav = jnp.where(keep, eav, 0.0)
                    ea_v[e] = eav
                    plsc.addupdate_scatter(esum_v, [cv[t] * H + lane], eav,
                                           mask=lane < H)
                return c2
            lax.fori_loop(0, KG // 16, gb, 0)
            pltpu.sync_copy(ea_v, ea_hbm.at[pl.ds(base, KG)])
            return c
        lax.fori_loop(0, CHG, chunk, 0)
        pltpu.sync_copy(esum_v, esum_hbm.at[wid])

    return body(asrc16, adst16, row3, col3)


def _sc_gat2(xh, ea16, esum16, row3, col3, zinit):
    npt = N // NS

    @functools.partial(
        pl.kernel,
        out_type=jax.ShapeDtypeStruct((NC, N, HID), jnp.float32),
        mesh=_SC_MESH,
        compiler_params=_SC_PARAMS,
        scratch_types=[
            pltpu.VMEM((CHG2, KG2), jnp.int32),
            pltpu.VMEM((CHG2, KG2), jnp.int32),
            pltpu.VMEM((2, KG2, H * HID), jnp.float32),
            pltpu.VMEM((2, KG2, 16), jnp.float32),
            pltpu.VMEM((2, KG2, 16), jnp.float32),
            pltpu.VMEM((KG2, HID), jnp.float32),
            pltpu.VMEM_SHARED((N, HID), jnp.float32),
            pltpu.SemaphoreType.DMA,
        ],
    )
    def body(xh_hbm, ea_hbm, esum_hbm, row_hbm, col_hbm, zinit_hbm, out_hbm,
             row_v, col_v, xh_v, ea_v, es_v, msg_v, acc_sh, sem):
        cid = lax.axis_index("c")
        sid = lax.axis_index("s")
        wid = _wid()
        pltpu.sync_copy(row_hbm.at[wid], row_v)
        pltpu.sync_copy(col_hbm.at[wid], col_v)
        pltpu.sync_copy(zinit_hbm.at[pl.ds(sid * npt, npt)],
                        acc_sh.at[pl.ds(sid * npt, npt)])
        plsc.subcore_barrier()

        def fire(k, b):
            base = wid * EG_PT + k * KG2
            pltpu.async_copy(ea_hbm.at[pl.ds(base, KG2)], ea_v.at[b], sem)
            pltpu.async_copy(xh_hbm.at[row_v.at[k]], xh_v.at[b], sem)
            pltpu.async_copy(esum_hbm.at[col_v.at[k]], es_v.at[b], sem)

        def drain(b):
            pltpu.make_async_copy(
                ea_hbm.at[pl.ds(0, KG2)], ea_v.at[b], sem).wait()
            pltpu.make_async_copy(
                xh_hbm.at[pl.ds(0, KG2)], xh_v.at[b], sem).wait()
            pltpu.make_async_copy(
                esum_hbm.at[pl.ds(0, KG2)], es_v.at[b], sem).wait()

        fire(0, 0)
        fire(1, 1)

        def pair(i, c):
            for b in range(2):
                k = 2 * i + b
                drain(b)

                def eb(e, c2):
                    att = ea_v[b, e] / es_v[b, e]
                    m = [None] * (HID // 16)
                    for h in range(H):
                        ah = jnp.full((16,), att[h], jnp.float32)
                        for j in range(HID // 16):
                            xv = xh_v[b, e, pl.ds(h * HID + j * 16, 16)]
                            m[j] = ah * xv if h == 0 else m[j] + ah * xv
                    for j in range(HID // 16):
                        msg_v[e, pl.ds(j * 16, 16)] = m[j]
                    return c2
                lax.fori_loop(0, KG2, eb, 0)
                pltpu.sync_copy(msg_v, acc_sh.at[col_v.at[k]], add=True)

                @pl.when(k + 2 < CHG2)
                def _():
                    fire(k + 2, b)
            return c
        lax.fori_loop(0, CHG2 // 2, pair, 0)
        plsc.subcore_barrier()
        pltpu.sync_copy(acc_sh.at[pl.ds(sid * npt, npt)],
                        out_hbm.at[cid, pl.ds(sid * npt, npt)])

    return body(xh, ea16, esum16, row3, col3, zinit)


def kernel(x, local_edge_index, global_edge_index, W_lp, b_lp, W_gcn, b_gcn,
           W_gat, att_src, att_dst, b_gat, W_pool, b_pool, W_c1, b_c1,
           W_c2, b_c2, W_u, b_u):
    row_l = local_edge_index[0]
    col_l = local_edge_index[1]
    pad = jnp.zeros((EGP - EG,), jnp.int32)
    row_g3 = jnp.concatenate([global_edge_index[0], pad]).reshape(NW, -1)
    col_g3 = jnp.concatenate([global_edge_index[1], pad]).reshape(NW, -1)
    zinit = jnp.zeros((N, HID), jnp.float32)

    deg_parts = _sc_deg(col_l.reshape(NW, EL_PT))
    deg_part = deg_parts.T

    asrT = att_src.reshape(H, HID)
    adrT = att_dst.reshape(H, HID)
    xwd, xh, asrc16, adst16, eself = _tc1(
        x, deg_part, W_lp, b_lp.reshape(1, HID), W_gcn, W_gat, asrT, adrT)

    acc_parts = _sc_gcn(xwd, row_l.reshape(NW, CHL, KL),
                        col_l.reshape(NW, CHL, KL), zinit)

    ea16, esum_pf = _sc_gat1(asrc16, adst16,
                             row_g3.reshape(NW, CHG, KG),
                             col_g3.reshape(NW, CHG, KG))
    esum_parts = esum_pf.reshape(NW, N, H)

    xlg, esum16, esum4 = _tc2(
        acc_parts, xwd, deg_part, b_gcn.reshape(1, HID), esum_parts, eself)

    msg_parts = _sc_gat2(xh, ea16, esum16,
                         row_g3.reshape(NW, CHG2, KG2),
                         col_g3.reshape(NW, CHG2, KG2), zinit)

    pred, unc = _tc3(xlg, msg_parts, xh, eself, esum4,
                     b_gat.reshape(1, HID), W_pool, b_pool.reshape(1, HID),
                     W_c1, b_c1.reshape(1, HID // 2), W_c2,
                     b_c2.reshape(1, OUT), W_u, b_u.reshape(1, OUT))
    return (pred, unc)


# speedup vs baseline: 227.6031x; 1.0288x over previous
import functools
import jax
import jax.numpy as jnp
from jax import lax
from jax.experimental import pallas as pl
from jax.experimental.pallas import tpu as pltpu
from jax.experimental.pallas import tpu_sc as plsc

N = 10000
D = 128
HID = 128
OUT = 16
EL = 320000
EG = 80000
H = 4

NC = 2
NS = 16
NW = NC * NS

BN = 1000
NGRID = N // BN
EGP = 81920
EG_PT = EGP // NW

PREC = lax.Precision.HIGHEST


def _dot(a, b):
    return jnp.dot(a, b, precision=PREC)


def _lrelu(v):
    return jnp.where(v > 0, v, 0.2 * v)


def _tc1_body(x_ref, degp_ref, wlp_ref, blp_ref, wgcn_ref, wgat_ref,
              asr_ref, adr_ref,
              xwd_o, xh_o, asrc16_o, adst16_o, eself_o):
    xb = x_ref[...]
    xl = jnp.maximum(_dot(xb, wlp_ref[...]) + blp_ref[...], 0.0)
    deg = jnp.sum(degp_ref[...], axis=1) + 1.0
    dis = lax.rsqrt(deg)
    xw = _dot(xl, wgcn_ref[...])
    xwd_o[...] = dis[:, None] * xw
    xh = _dot(xl, wgat_ref[...])
    xh_o[...] = xh
    acols = []
    bcols = []
    for h in range(H):
        xh_h = xh[:, h * HID:(h + 1) * HID]
        acols.append(_dot(xh_h, asr_ref[...][h][:, None]))
        bcols.append(_dot(xh_h, adr_ref[...][h][:, None]))
    a_src = jnp.concatenate(acols, axis=1)
    a_dst = jnp.concatenate(bcols, axis=1)
    zpad = jnp.zeros((a_src.shape[0], 16 - H), jnp.float32)
    asrc16_o[...] = jnp.concatenate([a_src, zpad], axis=1)
    adst16_o[...] = jnp.concatenate([a_dst, zpad], axis=1)
    eself_o[...] = jnp.exp(_lrelu(a_src + a_dst))


def _tc1(x, deg_part, W_lp, b_lp, W_gcn, W_gat, asrT, adrT):
    return pl.pallas_call(
        _tc1_body,
        grid=(NGRID,),
        in_specs=[
            pl.BlockSpec((BN, D), lambda i: (i, 0)),
            pl.BlockSpec((BN, NW), lambda i: (i, 0)),
            pl.BlockSpec((D, HID), lambda i: (0, 0)),
            pl.BlockSpec((1, HID), lambda i: (0, 0)),
            pl.BlockSpec((HID, HID), lambda i: (0, 0)),
            pl.BlockSpec((HID, H * HID), lambda i: (0, 0)),
            pl.BlockSpec((H, HID), lambda i: (0, 0)),
            pl.BlockSpec((H, HID), lambda i: (0, 0)),
        ],
        out_specs=[
            pl.BlockSpec((BN, HID), lambda i: (i, 0)),
            pl.BlockSpec((BN, H * HID), lambda i: (i, 0)),
            pl.BlockSpec((BN, 16), lambda i: (i, 0)),
            pl.BlockSpec((BN, 16), lambda i: (i, 0)),
            pl.BlockSpec((BN, H), lambda i: (i, 0)),
        ],
        out_shape=[
            jax.ShapeDtypeStruct((N, HID), jnp.float32),
            jax.ShapeDtypeStruct((N, H * HID), jnp.float32),
            jax.ShapeDtypeStruct((N, 16), jnp.float32),
            jax.ShapeDtypeStruct((N, 16), jnp.float32),
            jax.ShapeDtypeStruct((N, H), jnp.float32),
        ],
    )(x, deg_part, W_lp, b_lp, W_gcn, W_gat, asrT, adrT)


def _tc2_body(accp_ref, xwd_ref, degp_ref, bgcn_ref, esump_ref, eself_ref,
              xlg_o, esum16_o, esum4_o):
    deg = jnp.sum(degp_ref[...], axis=1) + 1.0
    dis = lax.rsqrt(deg)
    acc = accp_ref[...][0] + accp_ref[...][1] + xwd_ref[...]
    xlg_o[...] = jnp.maximum(dis[:, None] * acc + bgcn_ref[...], 0.0)
    esum = jnp.sum(esump_ref[...], axis=0) + eself_ref[...]
    esum4_o[...] = esum
    opad = jnp.ones((esum.shape[0], 16 - H), jnp.float32)
    esum16_o[...] = jnp.concatenate([esum, opad], axis=1)


def _tc2(acc_parts, xwd, deg_part, b_gcn, esum_parts, eself):
    return pl.pallas_call(
        _tc2_body,
        grid=(NGRID,),
        in_specs=[
            pl.BlockSpec((NC, BN, HID), lambda i: (0, i, 0)),
            pl.BlockSpec((BN, HID), lambda i: (i, 0)),
            pl.BlockSpec((BN, NW), lambda i: (i, 0)),
            pl.BlockSpec((1, HID), lambda i: (0, 0)),
            pl.BlockSpec((NW, BN, H), lambda i: (0, i, 0)),
            pl.BlockSpec((BN, H), lambda i: (i, 0)),
        ],
        out_specs=[
            pl.BlockSpec((BN, HID), lambda i: (i, 0)),
            pl.BlockSpec((BN, 16), lambda i: (i, 0)),
            pl.BlockSpec((BN, H), lambda i: (i, 0)),
        ],
        out_shape=[
            jax.ShapeDtypeStruct((N, HID), jnp.float32),
            jax.ShapeDtypeStruct((N, 16), jnp.float32),
            jax.ShapeDtypeStruct((N, H), jnp.float32),
        ],
    )(acc_parts, xwd, deg_part, b_gcn, esum_parts, eself)


def _tc3_body(xlg_ref, msgp_ref, xh_ref, eself_ref, esum4_ref, bgat_ref,
              wpa_ref, wpb_ref, bpool_ref, wc1_ref, bc1_ref, wc2_ref,
              bc2_ref, wu_ref, bu_ref,
              pred_o, unc_o, acc_s):
    i = pl.program_id(0)
    att_self = eself_ref[...] / esum4_ref[...]
    xh = xh_ref[...]
    self_msg = att_self[:, 0:1] * xh[:, 0:HID]
    for h in range(1, H):
        self_msg = self_msg + att_self[:, h:h + 1] * xh[:, h * HID:(h + 1) * HID]
    gat = jnp.maximum(
        (msgp_ref[...][0] + msgp_ref[...][1] + self_msg) * (1.0 / H)
        + bgat_ref[...], 0.0)
    pooled = jnp.maximum(
        _dot(xlg_ref[...], wpa_ref[...]) + _dot(gat, wpb_ref[...])
        + bpool_ref[...], 0.0)
    psum = jnp.sum(pooled, axis=0, keepdims=True)

    @pl.when(i == 0)
    def _():
        acc_s[...] = psum

    @pl.when(i > 0)
    def _():
        acc_s[...] = acc_s[...] + psum

    @pl.when(i == NGRID - 1)
    def _():
        xf = acc_s[...] * (1.0 / N)
        hh = jnp.maximum(_dot(xf, wc1_ref[...]) + bc1_ref[...], 0.0)
        pred_o[...] = _dot(hh, wc2_ref[...]) + bc2_ref[...]
        zu = _dot(xf, wu_ref[...]) + bu_ref[...]
        unc_o[...] = 1.0 / (1.0 + jnp.exp(-zu))


def _tc3(xlg, msg_parts, xh, eself, esum4, b_gat, W_pool, b_pool,
         W_c1, b_c1, W_c2, b_c2, W_u, b_u):
    wpa = W_pool[:HID]
    wpb = W_pool[HID:]
    return pl.pallas_call(
        _tc3_body,
        grid=(NGRID,),
        in_specs=[
            pl.BlockSpec((BN, HID), lambda i: (i, 0)),
            pl.BlockSpec((NC, BN, HID), lambda i: (0, i, 0)),
            pl.BlockSpec((BN, H * HID), lambda i: (i, 0)),
            pl.BlockSpec((BN, H), lambda i: (i, 0)),
            pl.BlockSpec((BN, H), lambda i: (i, 0)),
            pl.BlockSpec((1, HID), lambda i: (0, 0)),
            pl.BlockSpec((HID, HID), lambda i: (0, 0)),
            pl.BlockSpec((HID, HID), lambda i: (0, 0)),
            pl.BlockSpec((1, HID), lambda i: (0, 0)),
            pl.BlockSpec((HID, HID // 2), lambda i: (0, 0)),
            pl.BlockSpec((1, HID // 2), lambda i: (0, 0)),
            pl.BlockSpec((HID // 2, OUT), lambda i: (0, 0)),
            pl.BlockSpec((1, OUT), lambda i: (0, 0)),
            pl.BlockSpec((HID, OUT), lambda i: (0, 0)),
            pl.BlockSpec((1, OUT), lambda i: (0, 0)),
        ],
        out_specs=[
            pl.BlockSpec((1, OUT), lambda i: (0, 0)),
            pl.BlockSpec((1, OUT), lambda i: (0, 0)),
        ],
        out_shape=[
            jax.ShapeDtypeStruct((1, OUT), jnp.float32),
            jax.ShapeDtypeStruct((1, OUT), jnp.float32),
        ],
        scratch_shapes=[pltpu.VMEM((1, HID), jnp.float32)],
    )(xlg, msg_parts, xh, eself, esum4, b_gat, wpa, wpb, b_pool,
      W_c1, b_c1, W_c2, b_c2, W_u, b_u)


EL_PT = EL // NW
KL = 80
CHL = EL_PT // KL
KG = 640
CHG = EG_PT // KG
KG2 = 32
CHG2 = EG_PT // KG2

_SC_MESH = plsc.VectorSubcoreMesh(
    core_axis_name="c", subcore_axis_name="s",
    num_cores=NC, num_subcores=NS)
_SC_PARAMS = pltpu.CompilerParams(needs_layout_passes=False, use_tc_tiling_on_sc=False)


def _wid():
    return lax.axis_index("s") * NC + lax.axis_index("c")


def _sc_deg(col2):
    @functools.partial(
        pl.kernel,
        out_type=jax.ShapeDtypeStruct((NW, N), jnp.float32),
        mesh=_SC_MESH,
        compiler_params=_SC_PARAMS,
        scratch_types=[
            pltpu.VMEM((EL_PT,), jnp.int32),
            pltpu.VMEM((N,), jnp.float32),
        ],
    )
    def body(col_hbm, out_hbm, col_v, deg_v):
        wid = _wid()
        pltpu.sync_copy(col_hbm.at[wid], col_v)
        z16 = jnp.zeros((16,), jnp.float32)
        o16 = jnp.ones((16,), jnp.float32)

        def zb(i, c):
            deg_v[pl.ds(i * 16, 16)] = z16
            return c
        lax.fori_loop(0, N // 16, zb, 0)

        def eb(i, c):
            idx = col_v[pl.ds(i * 16, 16)]
            plsc.addupdate_scatter(deg_v, [idx], o16)
            return c
        lax.fori_loop(0, EL_PT // 16, eb, 0)
        pltpu.sync_copy(deg_v, out_hbm.at[wid])

    return body(col2)


def _sc_gcn(xwd, row3, col3, zinit):
    npt = N // NS

    @functools.partial(
        pl.kernel,
        out_type=jax.ShapeDtypeStruct((NC, N, HID), jnp.float32),
        mesh=_SC_MESH,
        compiler_params=_SC_PARAMS,
        scratch_types=[
            pltpu.VMEM((CHL, KL), jnp.int32),
            pltpu.VMEM((CHL, KL), jnp.int32),
            pltpu.VMEM((2, KL, HID), jnp.float32),
            pltpu.VMEM_SHARED((N, HID), jnp.float32),
            pltpu.SemaphoreType.DMA,
        ],
    )
    def body(xwd_hbm, row_hbm, col_hbm, zinit_hbm, out_hbm,
             row_v, col_v, rows_v, acc_sh, sem):
        cid = lax.axis_index("c")
        sid = lax.axis_index("s")
        wid = _wid()
        pltpu.sync_copy(row_hbm.at[wid], row_v)
        pltpu.sync_copy(col_hbm.at[wid], col_v)
        pltpu.sync_copy(zinit_hbm.at[pl.ds(sid * npt, npt)],
                        acc_sh.at[pl.ds(sid * npt, npt)])
        plsc.subcore_barrier()
        pltpu.async_copy(xwd_hbm.at[row_v.at[0]], rows_v.at[0], sem)
        pltpu.async_copy(xwd_hbm.at[row_v.at[1]], rows_v.at[1], sem)

        def pair(i, c):
            for b in range(2):
                k = 2 * i + b
                pltpu.make_async_copy(
                    xwd_hbm.at[pl.ds(0, KL)], rows_v.at[b], sem).wait()
                pltpu.sync_copy(rows_v.at[b], acc_sh.at[col_v.at[k]], add=True)

                @pl.when(k + 2 < CHL)
                def _():
                    pltpu.async_copy(xwd_hbm.at[row_v.at[k + 2]],
                                     rows_v.at[b], sem)
            return c
        lax.fori_loop(0, CHL // 2, pair, 0)
        if CHL % 2:
            k = CHL - 1
            b = k % 2
            pltpu.make_async_copy(
                xwd_hbm.at[pl.ds(0, KL)], rows_v.at[b], sem).wait()
            pltpu.sync_copy(rows_v.at[b], acc_sh.at[col_v.at[k]], add=True)
        plsc.subcore_barrier()
        pltpu.sync_copy(acc_sh.at[pl.ds(sid * npt, npt)],
                        out_hbm.at[cid, pl.ds(sid * npt, npt)])

    return body(xwd, row3, col3, zinit)


def _sc_gat1(asrc16, adst16, row3, col3):
    @functools.partial(
        pl.kernel,
        out_type=[
            jax.ShapeDtypeStruct((EGP, 16), jnp.float32),
            jax.ShapeDtypeStruct((NW, N * H), jnp.float32),
        ],
        mesh=_SC_MESH,
        compiler_params=_SC_PARAMS,
        scratch_types=[
            pltpu.VMEM((CHG, KG), jnp.int32),
            pltpu.VMEM((CHG, KG), jnp.int32),
            pltpu.VMEM((2, KG, 16), jnp.float32),
            pltpu.VMEM((2, KG, 16), jnp.float32),
            pltpu.VMEM((2, KG, 16), jnp.float32),
            pltpu.VMEM((N * H,), jnp.float32),
            pltpu.SemaphoreType.DMA,
            pltpu.SemaphoreType.DMA,
        ],
    )
    def body(asrc_hbm, adst_hbm, row_hbm, col_hbm, ea_hbm, esum_hbm,
             row_v, col_v, va, vb, ea_v, esum_v, sem, semw):
        wid = _wid()
        z16 = jnp.zeros((16,), jnp.float32)
        lane = lax.iota(jnp.int32, 16)
        pltpu.sync_copy(row_hbm.at[wid], row_v)
        pltpu.sync_copy(col_hbm.at[wid], col_v)

        def zb(i, c):
            esum_v[pl.ds(i * 16, 16)] = z16
            return c
        lax.fori_loop(0, (N * H) // 16, zb, 0)

        def fire(k, b):
            pltpu.async_copy(asrc_hbm.at[row_v.at[k]], va.at[b], sem)
            pltpu.async_copy(adst_hbm.at[col_v.at[k]], vb.at[b], sem)

        def drain_g(b):
            pltpu.make_async_copy(
                asrc_hbm.at[pl.ds(0, KG)], va.at[b], sem).wait()
            pltpu.make_async_copy(
                asrc_hbm.at[pl.ds(0, KG)], vb.at[b], sem).wait()

        fire(0, 0)
        fire(1, 1)
        for k in range(CHG):
            b = k % 2
            drain_g(b)
            if k >= 2:
                pltpu.make_async_copy(
                    ea_hbm.at[pl.ds(0, KG)], ea_v.at[b], semw).wait()
            base = wid * EG_PT + k * KG

            def gb(g, c2, _b=b, _base=base, _k=k):
                cv = col_v[_k, pl.ds(g * 16, 16)]
                for t in range(16):
                    e = g * 16 + t
                    alpha = va[_b, e] + vb[_b, e]
                    alpha = jnp.where(alpha > 0, alpha, 0.2 * alpha)
                    eav = jnp.exp(alpha)
                    valid = (_base + e) < EG
                    keep = jnp.logical_and(lane < H, valid)
                    eav = jnp.where(keep, eav, 0.0)
                    ea_v[_b, e] = eav
                    plsc.addupdate_scatter(esum_v, [cv[t] * H + lane], eav,
                                           mask=lane < H)
                return c2
            lax.fori_loop(0, KG // 16, gb, 0)
            pltpu.async_copy(ea_v.at[b], ea_hbm.at[pl.ds(base, KG)], semw)
            if k + 2 < CHG:
                fire(k + 2, b)
        for b in range(2):
            pltpu.make_async_copy(
                ea_hbm.at[pl.ds(0, KG)], ea_v.at[b], semw).wait()
        pltpu.sync_copy(esum_v, esum_hbm.at[wid])

    return body(asrc16, adst16, row3, col3)


def _sc_gat2(xh, ea16, esum16, row3, col3, zinit):
    npt = N // NS

    @functools.partial(
        pl.kernel,
        out_type=jax.ShapeDtypeStruct((NC, N, HID), jnp.float32),
        mesh=_SC_MESH,
        compiler_params=_SC_PARAMS,
        scratch_types=[
            pltpu.VMEM((CHG2, KG2), jnp.int32),
            pltpu.VMEM((CHG2, KG2), jnp.int32),
            pltpu.VMEM((2, KG2, H * HID), jnp.float32),
            pltpu.VMEM((2, KG2, 16), jnp.float32),
            pltpu.VMEM((2, KG2, 16), jnp.float32),
            pltpu.VMEM((2, KG2, HID), jnp.float32),
            pltpu.VMEM_SHARED((N, HID), jnp.float32),
            pltpu.SemaphoreType.DMA,
            pltpu.SemaphoreType.DMA,
        ],
    )
    def body(xh_hbm, ea_hbm, esum_hbm, row_hbm, col_hbm, zinit_hbm, out_hbm,
             row_v, col_v, xh_v, ea_v, es_v, msg_v, acc_sh, sem, sems):
        cid = lax.axis_index("c")
        sid = lax.axis_index("s")
        wid = _wid()
        pltpu.sync_copy(row_hbm.at[wid], row_v)
        pltpu.sync_copy(col_hbm.at[wid], col_v)
        pltpu.sync_copy(zinit_hbm.at[pl.ds(sid * npt, npt)],
                        acc_sh.at[pl.ds(sid * npt, npt)])
        plsc.subcore_barrier()

        def fire(k, b):
            base = wid * EG_PT + k * KG2
            pltpu.async_copy(ea_hbm.at[pl.ds(base, KG2)], ea_v.at[b], sem)
            pltpu.async_copy(xh_hbm.at[row_v.at[k]], xh_v.at[b], sem)
            pltpu.async_copy(esum_hbm.at[col_v.at[k]], es_v.at[b], sem)

        def drain_g(b):
            pltpu.make_async_copy(
                ea_hbm.at[pl.ds(0, KG2)], ea_v.at[b], sem).wait()
            pltpu.make_async_copy(
                xh_hbm.at[pl.ds(0, KG2)], xh_v.at[b], sem).wait()
            pltpu.make_async_copy(
                esum_hbm.at[pl.ds(0, KG2)], es_v.at[b], sem).wait()

        def drain_s(b):
            pltpu.make_async_copy(
                zinit_hbm.at[pl.ds(0, KG2)], msg_v.at[b], sems).wait()

        def compute(k, b):
            def eb(e, c2):
                att = ea_v[b, e] / es_v[b, e]
                m = [None] * (HID // 16)
                for h in range(H):
                    ah = jnp.full((16,), att[h], jnp.float32)
                    for j in range(HID // 16):
                        xv = xh_v[b, e, pl.ds(h * HID + j * 16, 16)]
                        m[j] = ah * xv if h == 0 else m[j] + ah * xv
                for j in range(HID // 16):
                    msg_v[b, e, pl.ds(j * 16, 16)] = m[j]
                return c2
            lax.fori_loop(0, KG2, eb, 0)
            pltpu.async_copy(msg_v.at[b], acc_sh.at[col_v.at[k]], sems,
                             add=True)

        fire(0, 0)
        fire(1, 1)
        for k in (0, 1):
            b = k
            drain_g(b)
            compute(k, b)
            fire(k + 2, b)

        def pair(i, c):
            for b in range(2):
                k = 2 * i + b
                drain_g(b)
                drain_s(b)
                compute(k, b)

                @pl.when(k + 2 < CHG2)
                def _():
                    fire(k + 2, b)
            return c
        lax.fori_loop(1, CHG2 // 2, pair, 0)
        drain_s(0)
        drain_s(1)
        plsc.subcore_barrier()
        pltpu.sync_copy(acc_sh.at[pl.ds(sid * npt, npt)],
                        out_hbm.at[cid, pl.ds(sid * npt, npt)])

    return body(xh, ea16, esum16, row3, col3, zinit)


def kernel(x, local_edge_index, global_edge_index, W_lp, b_lp, W_gcn, b_gcn,
           W_gat, att_src, att_dst, b_gat, W_pool, b_pool, W_c1, b_c1,
           W_c2, b_c2, W_u, b_u):
    row_l = local_edge_index[0]
    col_l = local_edge_index[1]
    pad = jnp.zeros((EGP - EG,), jnp.int32)
    row_g3 = jnp.concatenate([global_edge_index[0], pad]).reshape(NW, -1)
    col_g3 = jnp.concatenate([global_edge_index[1], pad]).reshape(NW, -1)
    zinit = jnp.zeros((N, HID), jnp.float32)

    deg_parts = _sc_deg(col_l.reshape(NW, EL_PT))
    deg_part = deg_parts.T

    asrT = att_src.reshape(H, HID)
    adrT = att_dst.reshape(H, HID)
    xwd, xh, asrc16, adst16, eself = _tc1(
        x, deg_part, W_lp, b_lp.reshape(1, HID), W_gcn, W_gat, asrT, adrT)

    acc_parts = _sc_gcn(xwd, row_l.reshape(NW, CHL, KL),
                        col_l.reshape(NW, CHL, KL), zinit)

    ea16, esum_pf = _sc_gat1(asrc16, adst16,
                             row_g3.reshape(NW, CHG, KG),
                             col_g3.reshape(NW, CHG, KG))
    esum_parts = esum_pf.reshape(NW, N, H)

    xlg, esum16, esum4 = _tc2(
        acc_parts, xwd, deg_part, b_gcn.reshape(1, HID), esum_parts, eself)

    msg_parts = _sc_gat2(xh, ea16, esum16,
                         row_g3.reshape(NW, CHG2, KG2),
                         col_g3.reshape(NW, CHG2, KG2), zinit)

    pred, unc = _tc3(xlg, msg_parts, xh, eself, esum4,
                     b_gat.reshape(1, HID), W_pool, b_pool.reshape(1, HID),
                     W_c1, b_c1.reshape(1, HID // 2), W_c2,
                     b_c2.reshape(1, OUT), W_u, b_u.reshape(1, OUT))
    return (pred, unc)
